# baseline (speedup 1.0000x reference)
import jax
import jax.numpy as jnp
from jax.experimental import pallas as pl

_B = 1024
_D = 768
_FF = 3072
_E = 8
_L = 6
_CAP = 160
_S = _E * _CAP
_EPAD = 128


def _rms_mul(h, w):
    return h * jax.lax.rsqrt(jnp.mean(h * h, axis=-1, keepdims=True) + 1e-6) * w


def _proj_kernel(x_ref, w_ref, b_ref, o_ref):
    o_ref[...] = jnp.dot(x_ref[...], w_ref[...],
                         preferred_element_type=jnp.float32) + b_ref[...]


def _attn_kernel(h_ref, g1_ref, wv_ref, wo_ref, g2_ref, ho_ref, n2_ref):
    h = h_ref[...]
    n = _rms_mul(h, g1_ref[...])
    t = jnp.dot(n, wv_ref[...], preferred_element_type=jnp.float32)
    o = jnp.dot(t, wo_ref[...], preferred_element_type=jnp.float32)
    hn = h + o
    ho_ref[...] = hn
    n2_ref[...] = _rms_mul(hn, g2_ref[...])


def _ffn_kernel(n2_ref, wi_ref, wo_ref, h_ref, o_ref):
    h1 = jnp.maximum(
        jnp.dot(n2_ref[...], wi_ref[...], preferred_element_type=jnp.float32),
        0.0)
    y = jnp.dot(h1, wo_ref[...], preferred_element_type=jnp.float32)
    o_ref[...] = h_ref[...] + y


def _router_kernel(n2_ref, rw_ref, ein_ref, pt_ref, gate_ref):
    n2 = n2_ref[...]
    logits = jnp.dot(n2, rw_ref[...], preferred_element_type=jnp.float32)
    col = jax.lax.broadcasted_iota(jnp.int32, (_B, _EPAD), 1)
    logits = jnp.where(col < _E, logits, jnp.float32(-1e30))
    m = jnp.max(logits, axis=-1, keepdims=True)
    ex = jnp.exp(logits - m)
    probs = ex / jnp.sum(ex, axis=-1, keepdims=True)
    gate = jnp.max(probs, axis=-1, keepdims=True)
    idx = jnp.min(jnp.where(probs == gate, col, _EPAD), axis=-1, keepdims=True)
    onehot = (col == idx).astype(jnp.float32)
    ri = jax.lax.broadcasted_iota(jnp.int32, (_B, _B), 0)
    ci = jax.lax.broadcasted_iota(jnp.int32, (_B, _B), 1)
    tril = (ci < ri).astype(jnp.float32)
    cnt = jnp.dot(tril, onehot, preferred_element_type=jnp.float32)
    pos = jnp.sum(cnt * onehot, axis=-1, keepdims=True).astype(jnp.int32)
    slot = jnp.where(pos < _CAP, idx * _CAP + pos, _S)
    scol = jax.lax.broadcasted_iota(jnp.int32, (_B, _S), 1)
    pt = (scol == slot).astype(jnp.float32)
    pt_ref[...] = pt
    gate_ref[...] = gate
    ein_ref[...] = jax.lax.dot_general(
        pt, n2, (((0,), (0,)), ((), ())), preferred_element_type=jnp.float32)


def _expert_kernel(ein_ref, wi_ref, wo_ref, h2_ref):
    h1 = jnp.maximum(
        jnp.dot(ein_ref[...], wi_ref[0], preferred_element_type=jnp.float32),
        0.0)
    h2_ref[...] = jnp.dot(h1, wo_ref[0], preferred_element_type=jnp.float32)


def _combine_kernel(pt_ref, h2_ref, gate_ref, h_ref, o_ref):
    y = jnp.dot(pt_ref[...], h2_ref[...], preferred_element_type=jnp.float32)
    o_ref[...] = h_ref[...] + gate_ref[...] * y


def _final_kernel(h_ref, g_ref, w_ref, b_ref, o_ref):
    n = _rms_mul(h_ref[...], g_ref[...])
    o_ref[...] = jnp.dot(n, w_ref[...],
                         preferred_element_type=jnp.float32) + b_ref[...]


def _pc(body, out_shape, **kw):
    return pl.pallas_call(body, out_shape=out_shape, **kw)


def kernel(x, proj_W, proj_b, attn_q, attn_k, attn_v, attn_o, ln1, ln2,
           router_W, moe_wi, moe_wo, ffn_wi, ffn_wo, final_ln, fc_W, fc_b):
    f32 = jnp.float32
    sd = jax.ShapeDtypeStruct
    xf = x.reshape(_B, -1)
    h = _pc(_proj_kernel, sd((_B, _D), f32))(xf, proj_W, proj_b.reshape(1, _D))
    for i in range(_L):
        g1 = ln1[i].reshape(1, _D)
        g2 = ln2[i].reshape(1, _D)
        h, n2 = _pc(_attn_kernel, (sd((_B, _D), f32), sd((_B, _D), f32)))(
            h, g1, attn_v[i], attn_o[i], g2)
        j = i // 2
        if i % 2 == 1:
            rw = jnp.zeros((_D, _EPAD), f32).at[:, :_E].set(router_W[j])
            ein, pt, gate = _pc(_router_kernel,
                                (sd((_S, _D), f32), sd((_B, _S), f32),
                                 sd((_B, 1), f32)))(n2, rw)
            h2 = pl.pallas_call(
                _expert_kernel,
                grid=(_E,),
                in_specs=[
                    pl.BlockSpec((_CAP, _D), lambda e: (e, 0)),
                    pl.BlockSpec((1, _D, _FF), lambda e: (e, 0, 0)),
                    pl.BlockSpec((1, _FF, _D), lambda e: (e, 0, 0)),
                ],
                out_specs=pl.BlockSpec((_CAP, _D), lambda e: (e, 0)),
                out_shape=sd((_S, _D), f32))(ein, moe_wi[j], moe_wo[j])
            h = _pc(_combine_kernel, sd((_B, _D), f32))(pt, h2, gate, h)
        else:
            h = _pc(_ffn_kernel, sd((_B, _D), f32))(n2, ffn_wi[j], ffn_wo[j], h)
    fw = jnp.zeros((_D, 128), f32).at[:, :10].set(fc_W)
    fb = jnp.zeros((1, 128), f32).at[:, :10].set(fc_b.reshape(1, 10))
    out = _pc(_final_kernel, sd((_B, 128), f32))(
        h, final_ln.reshape(1, _D), fw, fb)
    return out[:, :10]


# speedup vs baseline: 1.1013x; 1.1013x over previous
import jax
import jax.numpy as jnp
from jax.experimental import pallas as pl
from jax.experimental.pallas import tpu as pltpu

_B = 1024
_D = 768
_FF = 3072
_FH = _FF // 2
_E = 8
_L = 6
_CAP = 160
_S = _E * _CAP
_EPAD = 128


def _rms_mul(h, w):
    return h * jax.lax.rsqrt(jnp.mean(h * h, axis=-1, keepdims=True) + 1e-6) * w


def _dot(a, b):
    return jnp.dot(a, b, preferred_element_type=jnp.float32)


def _proj_kernel(x_ref, w_ref, b_ref, o_ref):
    k = pl.program_id(0)
    part = _dot(x_ref[...], w_ref[...])

    @pl.when(k == 0)
    def _():
        o_ref[...] = part + b_ref[...]

    @pl.when(k > 0)
    def _():
        o_ref[...] += part


def _dense_kernel(h_ref, g1_ref, wv_ref, wo_ref, g2_ref, wi_ref, w2_ref,
                  o_ref, n2_scr):
    t = pl.program_id(0)

    @pl.when(t == 0)
    def _():
        h = h_ref[...]
        n = _rms_mul(h, g1_ref[...])
        hn = h + _dot(_dot(n, wv_ref[...]), wo_ref[...])
        o_ref[...] = hn
        n2_scr[...] = _rms_mul(hn, g2_ref[...])

    @pl.when(t > 0)
    def _():
        h1 = jnp.maximum(_dot(n2_scr[...], wi_ref[...]), 0.0)
        o_ref[...] += _dot(h1, w2_ref[...])


def _moe_kernel(h_ref, g1_ref, wv_ref, wo_ref, g2_ref, rw_ref, wi_ref,
                w2_ref, o_ref, ein_scr, ptg_scr, h2_scr):
    t = pl.program_id(0)

    @pl.when(t == 0)
    def _():
        h = h_ref[...]
        n = _rms_mul(h, g1_ref[...])
        hn = h + _dot(_dot(n, wv_ref[...]), wo_ref[...])
        o_ref[...] = hn
        n2 = _rms_mul(hn, g2_ref[...])
        logits = _dot(n2, rw_ref[...])
        col = jax.lax.broadcasted_iota(jnp.int32, (_B, _EPAD), 1)
        logits = jnp.where(col < _E, logits, jnp.float32(-1e30))
        m = jnp.max(logits, axis=-1, keepdims=True)
        ex = jnp.exp(logits - m)
        probs = ex / jnp.sum(ex, axis=-1, keepdims=True)
        gate = jnp.max(probs, axis=-1, keepdims=True)
        idx = jnp.min(jnp.where(probs == gate, col, _EPAD), axis=-1,
                      keepdims=True)
        onehot = (col == idx).astype(jnp.float32)
        ri = jax.lax.broadcasted_iota(jnp.int32, (_B, _B), 0)
        ci = jax.lax.broadcasted_iota(jnp.int32, (_B, _B), 1)
        tril = (ci < ri).astype(jnp.float32)
        cnt = _dot(tril, onehot)
        pos = jnp.sum(cnt * onehot, axis=-1, keepdims=True).astype(jnp.int32)
        slot = jnp.where(pos < _CAP, idx * _CAP + pos, _S)
        scol = jax.lax.broadcasted_iota(jnp.int32, (_B, _S), 1)
        pt = (scol == slot).astype(jnp.float32)
        ptg_scr[...] = pt * gate
        ein_scr[...] = jax.lax.dot_general(
            pt, n2, (((0,), (0,)), ((), ())),
            preferred_element_type=jnp.float32)

    @pl.when((t >= 1) & (t <= 2 * _E))
    def _():
        tt = t - 1
        e = tt // 2
        f = tt % 2
        rows = ein_scr[pl.ds(e * _CAP, _CAP), :]
        h1 = jnp.maximum(_dot(rows, wi_ref[0]), 0.0)
        part = _dot(h1, w2_ref[0])

        @pl.when(f == 0)
        def _():
            h2_scr[pl.ds(e * _CAP, _CAP), :] = part

        @pl.when(f == 1)
        def _():
            h2_scr[pl.ds(e * _CAP, _CAP), :] += part

    @pl.when(t == 2 * _E + 1)
    def _():
        o_ref[...] += _dot(ptg_scr[...], h2_scr[...])


def _final_kernel(h_ref, g_ref, w_ref, b_ref, o_ref):
    n = _rms_mul(h_ref[...], g_ref[...])
    o_ref[...] = _dot(n, w_ref[...]) + b_ref[...]


def kernel(x, proj_W, proj_b, attn_q, attn_k, attn_v, attn_o, ln1, ln2,
           router_W, moe_wi, moe_wo, ffn_wi, ffn_wo, final_ln, fc_W, fc_b):
    f32 = jnp.float32
    sd = jax.ShapeDtypeStruct
    xf = x.reshape(_B, -1)

    h = pl.pallas_call(
        _proj_kernel,
        grid=(4,),
        in_specs=[
            pl.BlockSpec((_B, _D), lambda k: (0, k)),
            pl.BlockSpec((_D, _D), lambda k: (k, 0)),
            pl.BlockSpec((1, _D), lambda k: (0, 0)),
        ],
        out_specs=pl.BlockSpec((_B, _D), lambda k: (0, 0)),
        out_shape=sd((_B, _D), f32))(xf, proj_W, proj_b.reshape(1, _D))

    for i in range(_L):
        g1 = ln1[i].reshape(1, _D)
        g2 = ln2[i].reshape(1, _D)
        j = i // 2
        if i % 2 == 1:
            rw = jnp.zeros((_D, _EPAD), f32).at[:, :_E].set(router_W[j])

            def _wi_map(t):
                tt = jnp.clip(t - 1, 0, 2 * _E - 1)
                return (tt // 2, 0, tt % 2)

            def _w2_map(t):
                tt = jnp.clip(t - 1, 0, 2 * _E - 1)
                return (tt // 2, tt % 2, 0)

            h = pl.pallas_call(
                _moe_kernel,
                grid=(2 * _E + 2,),
                in_specs=[
                    pl.BlockSpec((_B, _D), lambda t: (0, 0)),
                    pl.BlockSpec((1, _D), lambda t: (0, 0)),
                    pl.BlockSpec((_D, _D), lambda t: (0, 0)),
                    pl.BlockSpec((_D, _D), lambda t: (0, 0)),
                    pl.BlockSpec((1, _D), lambda t: (0, 0)),
                    pl.BlockSpec((_D, _EPAD), lambda t: (0, 0)),
                    pl.BlockSpec((1, _D, _FH), _wi_map),
                    pl.BlockSpec((1, _FH, _D), _w2_map),
                ],
                out_specs=pl.BlockSpec((_B, _D), lambda t: (0, 0)),
                out_shape=sd((_B, _D), f32),
                scratch_shapes=[
                    pltpu.VMEM((_S, _D), f32),
                    pltpu.VMEM((_B, _S), f32),
                    pltpu.VMEM((_S, _D), f32),
                ])(h, g1, attn_v[i], attn_o[i], g2, rw, moe_wi[j], moe_wo[j])
        else:
            h = pl.pallas_call(
                _dense_kernel,
                grid=(5,),
                in_specs=[
                    pl.BlockSpec((_B, _D), lambda t: (0, 0)),
                    pl.BlockSpec((1, _D), lambda t: (0, 0)),
                    pl.BlockSpec((_D, _D), lambda t: (0, 0)),
                    pl.BlockSpec((_D, _D), lambda t: (0, 0)),
                    pl.BlockSpec((1, _D), lambda t: (0, 0)),
                    pl.BlockSpec((_D, _D),
                                 lambda t: (0, jnp.maximum(t - 1, 0))),
                    pl.BlockSpec((_D, _D),
                                 lambda t: (jnp.maximum(t - 1, 0), 0)),
                ],
                out_specs=pl.BlockSpec((_B, _D), lambda t: (0, 0)),
                out_shape=sd((_B, _D), f32),
                scratch_shapes=[pltpu.VMEM((_B, _D), f32)])(
                    h, g1, attn_v[i], attn_o[i], g2, ffn_wi[j], ffn_wo[j])

    fw = jnp.zeros((_D, 128), f32).at[:, :10].set(fc_W)
    fb = jnp.zeros((1, 128), f32).at[:, :10].set(fc_b.reshape(1, 10))
    out = pl.pallas_call(
        _final_kernel,
        out_shape=sd((_B, 128), f32))(h, final_ln.reshape(1, _D), fw, fb)
    return out[:, :10]
